# baseline (speedup 1.0000x reference)
import functools

import jax
import jax.numpy as jnp
from jax import lax
from jax.experimental import pallas as pl
from jax.experimental.pallas import tpu as pltpu

CNUM = 5
AUG = 4
LNUM = 3
H = 224
W = 224

R0 = 8
C0 = 128
PR = 240
PC = 384


def _sigm(x):
    return 1.0 / (1.0 + jnp.exp(-x))


def _tc_body(idx_ref, img_ref, lbl_ref, p_ref, w1_ref, b1_ref, w2_ref,
             b2t_ref, out_ref, pimg, ph, csc, stats):
    b = pl.program_id(0)
    i = pl.program_id(1)
    l = pl.program_id(2)

    @pl.when(jnp.logical_and(b == 0, jnp.logical_and(i == 0, l == 0)))
    def _zero_halo():
        pimg[...] = jnp.zeros((PR, PC), jnp.float32)
        ph[...] = jnp.zeros((4, PR, PC), jnp.float32)

    @pl.when(jnp.logical_and(i == 0, l == 0))
    def _init_sample():
        x = img_ref[0]
        mean = jnp.mean(x)
        var = jnp.mean((x - mean) * (x - mean))
        std = jnp.sqrt(var) + 1e-6
        stats[0] = mean
        stats[1] = std
        pimg[R0:R0 + H, C0:C0 + W] = (x - mean) / std

    mask = lbl_ref[0] == i

    @pl.when(l == 0)
    def _seed_c():
        csc[...] = jnp.where(mask, pimg[R0:R0 + H, C0:C0 + W], 0.0)

    taps = [
        pimg[R0 + ky - 1:R0 + ky - 1 + H, C0 + kx - 1:C0 + kx - 1 + W]
        for ky in range(3) for kx in range(3)
    ]
    for k in range(4):
        acc = jnp.full((H, W), b1_ref[0, 0, 0, l, k], jnp.float32)
        for ky in range(3):
            for kx in range(3):
                acc = acc + w1_ref[0, 0, 0, 0, k, 0, ky, kx] * taps[ky * 3 + kx]
        ph[k, R0:R0 + H, C0:C0 + W] = acc

    o = jnp.full((H, W), b2t_ref[0, 0, 0, l, 0], jnp.float32)
    for k in range(4):
        for ky in range(3):
            for kx in range(3):
                o = o + w2_ref[0, 0, 0, 0, 0, k, ky, kx] * ph[
                    k, R0 + ky - 1:R0 + ky - 1 + H, C0 + kx - 1:C0 + kx - 1 + W]
    mix = _sigm(o)

    p1 = _sigm(p_ref[0, 0, 0, l, 0])
    p2 = _sigm(p_ref[0, 0, 0, l, 1])
    p1v = _sigm(p_ref[0, 0, 0, l, 2])
    p2v = _sigm(p_ref[0, 0, 0, l, 3])
    c = csc[...]
    u = 1.0 - c
    uuc3 = 3.0 * u * u * c
    ucc3 = 3.0 * u * c * c
    ct = uuc3 * p1 + ucc3 * p2 + c * c * c
    cv = u * u * u + uuc3 * p1v + ucc3 * p2v
    cnew = jnp.clip(ct * mix + cv * (1.0 - mix), 0.0, 1.0)
    csc[...] = cnew

    @pl.when(l == LNUM - 1)
    def _merge():
        img = pimg[R0:R0 + H, C0:C0 + W]
        pimg[R0:R0 + H, C0:C0 + W] = jnp.where(mask, cnew, img)

    @pl.when(jnp.logical_and(i == CNUM - 1, l == LNUM - 1))
    def _emit():
        out_ref[0] = pimg[R0:R0 + H, C0:C0 + W] * stats[1] + stats[0]


def _tc_pipeline(index, img, lbl, param, conv_w1, conv_b1, conv_w2, conv_b2):
    B = img.shape[0]
    grid = (B, CNUM, LNUM)
    smem = functools.partial(pl.BlockSpec, memory_space=pltpu.SMEM)
    grid_spec = pltpu.PrefetchScalarGridSpec(
        num_scalar_prefetch=1,
        grid=grid,
        in_specs=[
            pl.BlockSpec((1, H, W), lambda b, i, l, idx: (b, 0, 0)),
            pl.BlockSpec((1, H, W), lambda b, i, l, idx: (b, 0, 0)),
            smem((1, 1, 1, LNUM, 7),
                 lambda b, i, l, idx: (idx[b], i, 0, 0, 0)),
            smem((1, 1, 1, 1, 4, 1, 3, 3),
                 lambda b, i, l, idx: (idx[b], i, 0, l, 0, 0, 0, 0)),
            smem((1, 1, 1, LNUM, 4),
                 lambda b, i, l, idx: (idx[b], i, 0, 0, 0)),
            smem((1, 1, 1, 1, 1, 4, 3, 3),
                 lambda b, i, l, idx: (idx[b], i, 0, l, 0, 0, 0, 0)),
            smem((1, 1, 1, LNUM, 1),
                 lambda b, i, l, idx: (idx[b], i, 0, 0, 0)),
        ],
        out_specs=pl.BlockSpec((1, H, W), lambda b, i, l, idx: (b, 0, 0)),
        scratch_shapes=[
            pltpu.VMEM((PR, PC), jnp.float32),
            pltpu.VMEM((4, PR, PC), jnp.float32),
            pltpu.VMEM((H, W), jnp.float32),
            pltpu.SMEM((2,), jnp.float32),
        ],
    )
    return pl.pallas_call(
        _tc_body,
        grid_spec=grid_spec,
        out_shape=jax.ShapeDtypeStruct((B, H, W), jnp.float32),
    )(index, img, lbl, param, conv_w1, conv_b1, conv_w2, conv_b2)


def kernel(GLA_img_aug, lbl, index, param, conv_w1, conv_b1, conv_w2, conv_b2):
    B = GLA_img_aug.shape[0]
    idx = index.astype(jnp.int32)
    out = _tc_pipeline(idx, GLA_img_aug.reshape(B, H, W), lbl, param,
                       conv_w1, conv_b1, conv_w2, conv_b2)
    return out.reshape(B, 1, H, W)


# speedup vs baseline: 6.9680x; 6.9680x over previous
import functools

import jax
import jax.numpy as jnp
from jax import lax
from jax.experimental import pallas as pl
from jax.experimental.pallas import tpu as pltpu

CNUM = 5
AUG = 4
LNUM = 3
H = 224
W = 224

R0 = 8
C0 = 128
PR = 240
PC = 384


def _sigm(x):
    return 1.0 / (1.0 + jnp.exp(-x))


def _tc_body(idx_ref, img_ref, lbl_ref, p_ref, w1_ref, b1_ref, w2_ref,
             b2t_ref, out_ref, pimg, ph, csc, stats):
    b = pl.program_id(0)
    i = pl.program_id(1)
    l = pl.program_id(2)

    @pl.when(jnp.logical_and(b == 0, jnp.logical_and(i == 0, l == 0)))
    def _zero_halo():
        pimg[...] = jnp.zeros((PR, PC), jnp.float32)
        ph[...] = jnp.zeros((4, PR, PC), jnp.float32)

    @pl.when(jnp.logical_and(i == 0, l == 0))
    def _init_sample():
        x = img_ref[0]
        mean = jnp.mean(x)
        var = jnp.mean((x - mean) * (x - mean))
        std = jnp.sqrt(var) + 1e-6
        stats[0] = mean
        stats[1] = std
        pimg[R0:R0 + H, C0:C0 + W] = (x - mean) / std

    mask = lbl_ref[0] == i

    @pl.when(l == 0)
    def _seed_c():
        csc[...] = jnp.where(mask, pimg[R0:R0 + H, C0:C0 + W], 0.0)

    taps = [
        pimg[R0 + ky - 1:R0 + ky - 1 + H, C0 + kx - 1:C0 + kx - 1 + W]
        for ky in range(3) for kx in range(3)
    ]
    w1o = i * (AUG * LNUM * 36) + l * 36
    b1o = i * (AUG * LNUM * 4) + l * 4
    w2o = i * (AUG * LNUM * 36) + l * 36
    b2o = i * (AUG * LNUM) + l
    po = i * (AUG * LNUM * 7) + l * 7
    for k in range(4):
        acc = jnp.full((H, W), b1_ref[0, 0, b1o + k], jnp.float32)
        for t in range(9):
            acc = acc + w1_ref[0, 0, w1o + k * 9 + t] * taps[t]
        ph[k, R0:R0 + H, C0:C0 + W] = acc

    o = jnp.full((H, W), b2t_ref[0, 0, b2o], jnp.float32)
    for k in range(4):
        for ky in range(3):
            for kx in range(3):
                o = o + w2_ref[0, 0, w2o + k * 9 + ky * 3 + kx] * ph[
                    k, R0 + ky - 1:R0 + ky - 1 + H, C0 + kx - 1:C0 + kx - 1 + W]
    mix = _sigm(o)

    p1 = _sigm(p_ref[0, 0, po + 0])
    p2 = _sigm(p_ref[0, 0, po + 1])
    p1v = _sigm(p_ref[0, 0, po + 2])
    p2v = _sigm(p_ref[0, 0, po + 3])
    c = csc[...]
    u = 1.0 - c
    uuc3 = 3.0 * u * u * c
    ucc3 = 3.0 * u * c * c
    ct = uuc3 * p1 + ucc3 * p2 + c * c * c
    cv = u * u * u + uuc3 * p1v + ucc3 * p2v
    cnew = jnp.clip(ct * mix + cv * (1.0 - mix), 0.0, 1.0)
    csc[...] = cnew

    @pl.when(l == LNUM - 1)
    def _merge():
        img = pimg[R0:R0 + H, C0:C0 + W]
        pimg[R0:R0 + H, C0:C0 + W] = jnp.where(mask, cnew, img)

    @pl.when(jnp.logical_and(i == CNUM - 1, l == LNUM - 1))
    def _emit():
        out_ref[0] = pimg[R0:R0 + H, C0:C0 + W] * stats[1] + stats[0]


def _tc_pipeline(index, img, lbl, param, conv_w1, conv_b1, conv_w2, conv_b2):
    B = img.shape[0]
    grid = (B, CNUM, LNUM)
    smem = functools.partial(pl.BlockSpec, memory_space=pltpu.SMEM)
    grid_spec = pltpu.PrefetchScalarGridSpec(
        num_scalar_prefetch=1,
        grid=grid,
        in_specs=[
            pl.BlockSpec((1, H, W), lambda b, i, l, idx: (b, 0, 0)),
            pl.BlockSpec((1, H, W), lambda b, i, l, idx: (b, 0, 0)),
            smem((1, 1, CNUM * AUG * LNUM * 7),
                 lambda b, i, l, idx: (idx[b], 0, 0)),
            smem((1, 1, CNUM * AUG * LNUM * 36),
                 lambda b, i, l, idx: (idx[b], 0, 0)),
            smem((1, 1, CNUM * AUG * LNUM * 4),
                 lambda b, i, l, idx: (idx[b], 0, 0)),
            smem((1, 1, CNUM * AUG * LNUM * 36),
                 lambda b, i, l, idx: (idx[b], 0, 0)),
            smem((1, 1, CNUM * AUG * LNUM),
                 lambda b, i, l, idx: (idx[b], 0, 0)),
        ],
        out_specs=pl.BlockSpec((1, H, W), lambda b, i, l, idx: (b, 0, 0)),
        scratch_shapes=[
            pltpu.VMEM((PR, PC), jnp.float32),
            pltpu.VMEM((4, PR, PC), jnp.float32),
            pltpu.VMEM((H, W), jnp.float32),
            pltpu.SMEM((2,), jnp.float32),
        ],
    )
    return pl.pallas_call(
        _tc_body,
        grid_spec=grid_spec,
        out_shape=jax.ShapeDtypeStruct((B, H, W), jnp.float32),
    )(index, img, lbl, param, conv_w1, conv_b1, conv_w2, conv_b2)


def kernel(GLA_img_aug, lbl, index, param, conv_w1, conv_b1, conv_w2, conv_b2):
    B = GLA_img_aug.shape[0]
    D = param.shape[0]
    idx = index.astype(jnp.int32)
    out = _tc_pipeline(idx, GLA_img_aug.reshape(B, H, W), lbl,
                       param.reshape(D, 1, -1), conv_w1.reshape(D, 1, -1),
                       conv_b1.reshape(D, 1, -1), conv_w2.reshape(D, 1, -1),
                       conv_b2.reshape(D, 1, -1))
    return out.reshape(B, 1, H, W)


# speedup vs baseline: 7.3983x; 1.0618x over previous
import functools

import jax
import jax.numpy as jnp
from jax import lax
from jax.experimental import pallas as pl
from jax.experimental.pallas import tpu as pltpu

CNUM = 5
AUG = 4
LNUM = 3
H = 224
W = 224

R0 = 8
C0 = 128
PR = 240
PC = 384


def _sigm(x):
    return 1.0 / (1.0 + jnp.exp(-x))


def _tc_body(idx_ref, img_ref, lbl_ref, p_ref, w1_ref, b1_ref, w2_ref,
             b2t_ref, out_ref, pimg, ph, csc, stats):
    b = pl.program_id(0)
    i = pl.program_id(1)
    l = pl.program_id(2)

    @pl.when(jnp.logical_and(b == 0, jnp.logical_and(i == 0, l == 0)))
    def _zero_halo():
        pimg[...] = jnp.zeros((PR, PC), jnp.float32)
        ph[...] = jnp.zeros((4, PR, PC), jnp.float32)

    @pl.when(jnp.logical_and(i == 0, l == 0))
    def _init_sample():
        x = img_ref[0]
        mean = jnp.mean(x)
        var = jnp.mean((x - mean) * (x - mean))
        std = jnp.sqrt(var) + 1e-6
        stats[0] = mean
        stats[1] = std
        pimg[R0:R0 + H, C0:C0 + W] = (x - mean) / std

    mask = lbl_ref[0] == i

    @pl.when(l == 0)
    def _seed_c():
        csc[...] = jnp.where(mask, pimg[R0:R0 + H, C0:C0 + W], 0.0)

    taps = [
        pimg[R0 + ky - 1:R0 + ky - 1 + H, C0 + kx - 1:C0 + kx - 1 + W]
        for ky in range(3) for kx in range(3)
    ]
    w1o = i * (AUG * LNUM * 36) + l * 36
    b1o = i * (LNUM * 4) + l * 4
    w2o = i * (AUG * LNUM * 36) + l * 36
    b2o = i * LNUM + l
    po = i * (LNUM * 4) + l * 4
    for k in range(4):
        acc = jnp.full((H, W), b1_ref[0, 0, b1o + k], jnp.float32)
        for t in range(9):
            acc = acc + w1_ref[0, 0, w1o + k * 9 + t] * taps[t]
        ph[k, R0:R0 + H, C0:C0 + W] = acc

    o = jnp.full((H, W), b2t_ref[0, 0, b2o], jnp.float32)
    for k in range(4):
        for ky in range(3):
            for kx in range(3):
                o = o + w2_ref[0, 0, w2o + k * 9 + ky * 3 + kx] * ph[
                    k, R0 + ky - 1:R0 + ky - 1 + H, C0 + kx - 1:C0 + kx - 1 + W]
    mix = _sigm(o)

    p1 = _sigm(p_ref[0, 0, po + 0])
    p2 = _sigm(p_ref[0, 0, po + 1])
    p1v = _sigm(p_ref[0, 0, po + 2])
    p2v = _sigm(p_ref[0, 0, po + 3])
    c = csc[...]
    u = 1.0 - c
    uuc3 = 3.0 * u * u * c
    ucc3 = 3.0 * u * c * c
    ct = uuc3 * p1 + ucc3 * p2 + c * c * c
    cv = u * u * u + uuc3 * p1v + ucc3 * p2v
    cnew = jnp.clip(ct * mix + cv * (1.0 - mix), 0.0, 1.0)
    csc[...] = cnew

    @pl.when(l == LNUM - 1)
    def _merge():
        img = pimg[R0:R0 + H, C0:C0 + W]
        pimg[R0:R0 + H, C0:C0 + W] = jnp.where(mask, cnew, img)

    @pl.when(jnp.logical_and(i == CNUM - 1, l == LNUM - 1))
    def _emit():
        out_ref[0] = pimg[R0:R0 + H, C0:C0 + W] * stats[1] + stats[0]


def _tc_pipeline(index, img, lbl, param, conv_w1, conv_b1, conv_w2, conv_b2):
    B = img.shape[0]
    grid = (B, CNUM, LNUM)
    smem = functools.partial(pl.BlockSpec, memory_space=pltpu.SMEM)
    grid_spec = pltpu.PrefetchScalarGridSpec(
        num_scalar_prefetch=1,
        grid=grid,
        in_specs=[
            pl.BlockSpec((1, H, W), lambda b, i, l, idx: (b, 0, 0)),
            pl.BlockSpec((1, H, W), lambda b, i, l, idx: (b, 0, 0)),
            smem((1, 1, CNUM * LNUM * 4),
                 lambda b, i, l, idx: (idx[b], 0, 0)),
            smem((1, 1, CNUM * AUG * LNUM * 36),
                 lambda b, i, l, idx: (idx[b], 0, 0)),
            smem((1, 1, CNUM * LNUM * 4),
                 lambda b, i, l, idx: (idx[b], 0, 0)),
            smem((1, 1, CNUM * AUG * LNUM * 36),
                 lambda b, i, l, idx: (idx[b], 0, 0)),
            smem((1, 1, CNUM * LNUM),
                 lambda b, i, l, idx: (idx[b], 0, 0)),
        ],
        out_specs=pl.BlockSpec((1, H, W), lambda b, i, l, idx: (b, 0, 0)),
        scratch_shapes=[
            pltpu.VMEM((PR, PC), jnp.float32),
            pltpu.VMEM((4, PR, PC), jnp.float32),
            pltpu.VMEM((H, W), jnp.float32),
            pltpu.SMEM((2,), jnp.float32),
        ],
    )
    return pl.pallas_call(
        _tc_body,
        grid_spec=grid_spec,
        out_shape=jax.ShapeDtypeStruct((B, H, W), jnp.float32),
    )(index, img, lbl, param, conv_w1, conv_b1, conv_w2, conv_b2)


def kernel(GLA_img_aug, lbl, index, param, conv_w1, conv_b1, conv_w2, conv_b2):
    B = GLA_img_aug.shape[0]
    D = param.shape[0]
    idx = index.astype(jnp.int32)
    pr = param[:, :, 0, :, :4].reshape(D, 1, CNUM * LNUM * 4)
    b1r = conv_b1[:, :, 0].reshape(D, 1, CNUM * LNUM * 4)
    b2r = conv_b2[:, :, 0].reshape(D, 1, CNUM * LNUM)
    out = _tc_pipeline(idx, GLA_img_aug.reshape(B, H, W), lbl,
                       pr, conv_w1.reshape(D, 1, -1),
                       b1r, conv_w2.reshape(D, 1, -1), b2r)
    return out.reshape(B, 1, H, W)
